# initial kernel scaffold (unmeasured)
import jax
import jax.numpy as jnp
from jax import lax
from jax.experimental import pallas as pl
from jax.experimental.pallas import tpu as pltpu

N_DEV = 16
B, SQ, SKV = 2, 512, 512
DH = 64
H_PER = 8
DM = 768
DHID = H_PER * DH
ROWS = B * SQ
CHUNK = ROWS // N_DEV


def _attn_body(x_ref, wq_ref, k_ref, v_ref, wo_ref, out_ref):
    h = pl.program_id(1)

    q = jnp.dot(x_ref[0], wq_ref[...], preferred_element_type=jnp.float32)
    k = k_ref[0, :, 0, :]
    scores = lax.dot_general(
        q, k, (((1,), (1,)), ((), ())), preferred_element_type=jnp.float32
    ) * 0.125

    qi = lax.broadcasted_iota(jnp.int32, (SQ, SKV), 0)
    ki = lax.broadcasted_iota(jnp.int32, (SQ, SKV), 1)
    mask = (jnp.abs(qi - ki) <= 128) | (ki < 32) | (qi < 32)
    scores = jnp.where(mask, scores, jnp.float32(-1e9))

    m = jnp.max(scores, axis=-1, keepdims=True)
    w = jnp.exp(scores - m)
    w = w / jnp.sum(w, axis=-1, keepdims=True)

    ctx = jnp.dot(w, v_ref[0, :, 0, :], preferred_element_type=jnp.float32)
    po = jnp.dot(ctx, wo_ref[...], preferred_element_type=jnp.float32)

    @pl.when(h == 0)
    def _():
        out_ref[...] = po

    @pl.when(h != 0)
    def _():
        out_ref[...] += po


def _allreduce_body(p_ref, out_ref, sbuf, rbuf, own_buf, agbuf,
                    rs_ssem, rs_rsem, ag_ssem, ag_rsem):
    i = lax.axis_index("i")
    left = lax.rem(i + N_DEV - 1, N_DEV)
    right = lax.rem(i + 1, N_DEV)

    barrier = pltpu.get_barrier_semaphore()
    for nbr in (left, right):
        pl.semaphore_signal(barrier, inc=1, device_id=(nbr,),
                            device_id_type=pl.DeviceIdType.MESH)
    pl.semaphore_wait(barrier, 2)

    for s in range(N_DEV - 1):
        c = lax.rem(i - s + 2 * N_DEV, N_DEV)
        rows = pl.ds(c * CHUNK, CHUNK)
        if s == 0:
            sbuf[s, :, :] = p_ref[rows, :]
        else:
            sbuf[s, :, :] = rbuf[s - 1, :, :] + p_ref[rows, :]
        rdma = pltpu.make_async_remote_copy(
            src_ref=sbuf.at[s],
            dst_ref=rbuf.at[s],
            send_sem=rs_ssem.at[s],
            recv_sem=rs_rsem.at[s],
            device_id=(right,),
            device_id_type=pl.DeviceIdType.MESH,
        )
        rdma.start()
        rdma.wait()

    own = lax.rem(i + 1, N_DEV)
    own_buf[...] = rbuf[N_DEV - 2, :, :] + p_ref[pl.ds(own * CHUNK, CHUNK), :]
    out_ref[pl.ds(own * CHUNK, CHUNK), :] = own_buf[...]

    for t in range(N_DEV - 1):
        rdma = pltpu.make_async_remote_copy(
            src_ref=own_buf if t == 0 else agbuf.at[t - 1],
            dst_ref=agbuf.at[t],
            send_sem=ag_ssem.at[t],
            recv_sem=ag_rsem.at[t],
            device_id=(right,),
            device_id_type=pl.DeviceIdType.MESH,
        )
        rdma.start()
        rdma.wait()
        g = lax.rem(i - t + 2 * N_DEV, N_DEV)
        out_ref[pl.ds(g * CHUNK, CHUNK), :] = agbuf[t, :, :]


def kernel(x, Wq, K_ext, V_ext, Wo):
    i = lax.axis_index("i")
    wq_s = lax.dynamic_slice(Wq, (0, i * DHID), (DM, DHID))
    wo_s = lax.dynamic_slice(Wo, (i * DHID, 0), (DHID, DM))

    partial = pl.pallas_call(
        _attn_body,
        grid=(B, H_PER),
        in_specs=[
            pl.BlockSpec((1, SQ, DM), lambda b, h: (b, 0, 0)),
            pl.BlockSpec((DM, DH), lambda b, h: (0, h)),
            pl.BlockSpec((1, SKV, 1, DH), lambda b, h: (b, 0, h, 0)),
            pl.BlockSpec((1, SKV, 1, DH), lambda b, h: (b, 0, h, 0)),
            pl.BlockSpec((DH, DM), lambda b, h: (h, 0)),
        ],
        out_specs=pl.BlockSpec((SQ, DM), lambda b, h: (b, 0)),
        out_shape=jax.ShapeDtypeStruct((ROWS, DM), jnp.float32),
    )(x, wq_s, K_ext, V_ext, wo_s)

    out = pl.pallas_call(
        _allreduce_body,
        out_shape=jax.ShapeDtypeStruct((ROWS, DM), jnp.float32),
        in_specs=[pl.BlockSpec(memory_space=pltpu.VMEM)],
        out_specs=pl.BlockSpec(memory_space=pltpu.VMEM),
        scratch_shapes=[
            pltpu.VMEM((N_DEV - 1, CHUNK, DM), jnp.float32),
            pltpu.VMEM((N_DEV - 1, CHUNK, DM), jnp.float32),
            pltpu.VMEM((CHUNK, DM), jnp.float32),
            pltpu.VMEM((N_DEV - 1, CHUNK, DM), jnp.float32),
            pltpu.SemaphoreType.DMA((N_DEV - 1,)),
            pltpu.SemaphoreType.DMA((N_DEV - 1,)),
            pltpu.SemaphoreType.DMA((N_DEV - 1,)),
            pltpu.SemaphoreType.DMA((N_DEV - 1,)),
        ],
        compiler_params=pltpu.CompilerParams(collective_id=0),
    )(partial)

    return out.reshape(B, SQ, DM)


# baseline (device time: 160309 ns/iter reference)
import jax
import jax.numpy as jnp
from jax import lax
from jax.experimental import pallas as pl
from jax.experimental.pallas import tpu as pltpu

N_DEV = 16
B, SQ, SKV = 2, 512, 512
DH = 64
H_PER = 8
DM = 768
DHID = H_PER * DH
ROWS = B * SQ
CHUNK = ROWS // N_DEV


def _attn_body(x_ref, wq_ref, k_ref, v_ref, wo_ref, out_ref):
    h = pl.program_id(1)

    q = jnp.dot(x_ref[0], wq_ref[0], preferred_element_type=jnp.float32)
    k = k_ref[0, 0]
    scores = lax.dot_general(
        q, k, (((1,), (1,)), ((), ())), preferred_element_type=jnp.float32
    ) * 0.125

    qi = lax.broadcasted_iota(jnp.int32, (SQ, SKV), 0)
    ki = lax.broadcasted_iota(jnp.int32, (SQ, SKV), 1)
    mask = (jnp.abs(qi - ki) <= 128) | (ki < 32) | (qi < 32)
    scores = jnp.where(mask, scores, jnp.float32(-1e9))

    m = jnp.max(scores, axis=-1, keepdims=True)
    w = jnp.exp(scores - m)
    w = w / jnp.sum(w, axis=-1, keepdims=True)

    ctx = jnp.dot(w, v_ref[0, 0], preferred_element_type=jnp.float32)
    po = jnp.dot(ctx, wo_ref[0], preferred_element_type=jnp.float32)

    @pl.when(h == 0)
    def _():
        out_ref[...] = po

    @pl.when(h != 0)
    def _():
        out_ref[...] += po


def _allreduce_body(p_ref, out_ref, sbuf, rbuf, own_buf, agbuf,
                    rs_ssem, rs_rsem, ag_ssem, ag_rsem):
    i = lax.axis_index("i")
    left = lax.rem(i + N_DEV - 1, N_DEV)
    right = lax.rem(i + 1, N_DEV)

    barrier = pltpu.get_barrier_semaphore()
    for nbr in (left, right):
        pl.semaphore_signal(barrier, inc=1, device_id=(nbr,),
                            device_id_type=pl.DeviceIdType.MESH)
    pl.semaphore_wait(barrier, 2)

    for s in range(N_DEV - 1):
        c = lax.rem(i - s + 2 * N_DEV, N_DEV)
        rows = pl.ds(c * CHUNK, CHUNK)
        if s == 0:
            sbuf[s, :, :] = p_ref[rows, :]
        else:
            sbuf[s, :, :] = rbuf[s - 1, :, :] + p_ref[rows, :]
        rdma = pltpu.make_async_remote_copy(
            src_ref=sbuf.at[s],
            dst_ref=rbuf.at[s],
            send_sem=rs_ssem.at[s],
            recv_sem=rs_rsem.at[s],
            device_id=(right,),
            device_id_type=pl.DeviceIdType.MESH,
        )
        rdma.start()
        rdma.wait()

    own = lax.rem(i + 1, N_DEV)
    own_buf[...] = rbuf[N_DEV - 2, :, :] + p_ref[pl.ds(own * CHUNK, CHUNK), :]
    out_ref[pl.ds(own * CHUNK, CHUNK), :] = own_buf[...]

    for t in range(N_DEV - 1):
        rdma = pltpu.make_async_remote_copy(
            src_ref=own_buf if t == 0 else agbuf.at[t - 1],
            dst_ref=agbuf.at[t],
            send_sem=ag_ssem.at[t],
            recv_sem=ag_rsem.at[t],
            device_id=(right,),
            device_id_type=pl.DeviceIdType.MESH,
        )
        rdma.start()
        rdma.wait()
        g = lax.rem(i - t + 2 * N_DEV, N_DEV)
        out_ref[pl.ds(g * CHUNK, CHUNK), :] = agbuf[t, :, :]


def kernel(x, Wq, K_ext, V_ext, Wo):
    i = lax.axis_index("i")
    wq_s = lax.dynamic_slice(Wq, (0, i * DHID), (DM, DHID))
    wo_s = lax.dynamic_slice(Wo, (i * DHID, 0), (DHID, DM))

    wq_r = wq_s.reshape(DM, H_PER, DH).transpose(1, 0, 2)
    wo_r = wo_s.reshape(H_PER, DH, DM)
    k_r = K_ext.transpose(2, 0, 1, 3)
    v_r = V_ext.transpose(2, 0, 1, 3)

    partial = pl.pallas_call(
        _attn_body,
        grid=(B, H_PER),
        in_specs=[
            pl.BlockSpec((1, SQ, DM), lambda b, h: (b, 0, 0)),
            pl.BlockSpec((1, DM, DH), lambda b, h: (h, 0, 0)),
            pl.BlockSpec((1, 1, SKV, DH), lambda b, h: (h, b, 0, 0)),
            pl.BlockSpec((1, 1, SKV, DH), lambda b, h: (h, b, 0, 0)),
            pl.BlockSpec((1, DH, DM), lambda b, h: (h, 0, 0)),
        ],
        out_specs=pl.BlockSpec((SQ, DM), lambda b, h: (b, 0)),
        out_shape=jax.ShapeDtypeStruct((ROWS, DM), jnp.float32),
    )(x, wq_r, k_r, v_r, wo_r)

    out = pl.pallas_call(
        _allreduce_body,
        out_shape=jax.ShapeDtypeStruct((ROWS, DM), jnp.float32),
        in_specs=[pl.BlockSpec(memory_space=pltpu.VMEM)],
        out_specs=pl.BlockSpec(memory_space=pltpu.VMEM),
        scratch_shapes=[
            pltpu.VMEM((N_DEV - 1, CHUNK, DM), jnp.float32),
            pltpu.VMEM((N_DEV - 1, CHUNK, DM), jnp.float32),
            pltpu.VMEM((CHUNK, DM), jnp.float32),
            pltpu.VMEM((N_DEV - 1, CHUNK, DM), jnp.float32),
            pltpu.SemaphoreType.DMA((N_DEV - 1,)),
            pltpu.SemaphoreType.DMA((N_DEV - 1,)),
            pltpu.SemaphoreType.DMA((N_DEV - 1,)),
            pltpu.SemaphoreType.DMA((N_DEV - 1,)),
        ],
        compiler_params=pltpu.CompilerParams(collective_id=0),
    )(partial)

    return out.reshape(B, SQ, DM)


# device time: 107692 ns/iter; 1.4886x vs baseline; 1.4886x over previous
import jax
import jax.numpy as jnp
from jax import lax
from jax.experimental import pallas as pl
from jax.experimental.pallas import tpu as pltpu

N_DEV = 16
B, SQ, SKV = 2, 512, 512
DH = 64
H_PER = 8
DM = 768
DHID = H_PER * DH
ROWS = B * SQ
CHUNK = ROWS // N_DEV
N_GROUP = 4
GROWS = ROWS // N_GROUP
CPG = N_DEV // N_GROUP

_MESH = pl.DeviceIdType.MESH


def _fused_body(x_ref, wq_ref, k_ref, v_ref, wo_ref, out_ref,
                sbuf, rbuf, agbuf, own_buf,
                rs_ssem, rs_rsem, ag_ssem, ag_rsem):
    i = lax.axis_index("i")

    barrier = pltpu.get_barrier_semaphore()
    for p in range(N_DEV):
        pl.semaphore_signal(barrier, inc=1, device_id=(p,),
                            device_id_type=_MESH)
    pl.semaphore_wait(barrier, N_DEV)

    for g in range(N_GROUP):
        b = g // (N_GROUP // B)
        sq_off = (g % (N_GROUP // B)) * GROWS
        x_g = x_ref[g * GROWS:(g + 1) * GROWS, :]

        qi = sq_off + lax.broadcasted_iota(jnp.int32, (GROWS, SKV), 0)
        ki = lax.broadcasted_iota(jnp.int32, (GROWS, SKV), 1)
        mask = (jnp.abs(qi - ki) <= 128) | (ki < 32) | (qi < 32)

        po = jnp.zeros((GROWS, DM), jnp.float32)
        for h in range(H_PER):
            q = jnp.dot(x_g, wq_ref[h], preferred_element_type=jnp.float32)
            s = lax.dot_general(
                q, k_ref[h, b], (((1,), (1,)), ((), ())),
                preferred_element_type=jnp.float32) * 0.125
            s = jnp.where(mask, s, jnp.float32(-1e9))
            m = jnp.max(s, axis=-1, keepdims=True)
            w = jnp.exp(s - m)
            w = w / jnp.sum(w, axis=-1, keepdims=True)
            ctx = jnp.dot(w, v_ref[h, b], preferred_element_type=jnp.float32)
            po = po + jnp.dot(ctx, wo_ref[h], preferred_element_type=jnp.float32)

        for u in range(CPG):
            c = g * CPG + u
            sbuf[c, :, :] = po[u * CHUNK:(u + 1) * CHUNK, :]

            @pl.when(c != i)
            def _(c=c):
                rdma = pltpu.make_async_remote_copy(
                    src_ref=sbuf.at[c],
                    dst_ref=rbuf.at[i],
                    send_sem=rs_ssem.at[c],
                    recv_sem=rs_rsem.at[i],
                    device_id=(c,), device_id_type=_MESH,
                )
                rdma.start()

    own = sbuf[pl.ds(i, 1), :, :].reshape(CHUNK, DM)
    acc = None
    for k in range(N_DEV):
        @pl.when(k != i)
        def _(k=k):
            pltpu.make_async_remote_copy(
                src_ref=sbuf.at[k], dst_ref=rbuf.at[k],
                send_sem=rs_ssem.at[k], recv_sem=rs_rsem.at[k],
                device_id=(0,), device_id_type=_MESH,
            ).wait_recv()
        contrib = jnp.where(k == i, own, rbuf[k, :, :])
        acc = contrib if acc is None else acc + contrib
    own_buf[...] = acc

    for j in range(1, N_DEV):
        t = lax.rem(i + j, N_DEV)
        rdma = pltpu.make_async_remote_copy(
            src_ref=own_buf,
            dst_ref=agbuf.at[i],
            send_sem=ag_ssem.at[j - 1],
            recv_sem=ag_rsem.at[i],
            device_id=(t,), device_id_type=_MESH,
        )
        rdma.start()

    for k in range(N_DEV):
        @pl.when(k != i)
        def _(k=k):
            pltpu.make_async_remote_copy(
                src_ref=own_buf, dst_ref=agbuf.at[k],
                send_sem=ag_ssem.at[0], recv_sem=ag_rsem.at[k],
                device_id=(0,), device_id_type=_MESH,
            ).wait_recv()
        out_ref[k * CHUNK:(k + 1) * CHUNK, :] = jnp.where(
            k == i, acc, agbuf[k, :, :])

    for c in range(N_DEV):
        @pl.when(c != i)
        def _(c=c):
            pltpu.make_async_remote_copy(
                src_ref=sbuf.at[c], dst_ref=rbuf.at[0],
                send_sem=rs_ssem.at[c], recv_sem=rs_rsem.at[0],
                device_id=(0,), device_id_type=_MESH,
            ).wait_send()
    for j in range(1, N_DEV):
        pltpu.make_async_remote_copy(
            src_ref=own_buf, dst_ref=agbuf.at[0],
            send_sem=ag_ssem.at[j - 1], recv_sem=ag_rsem.at[0],
            device_id=(0,), device_id_type=_MESH,
        ).wait_send()


def kernel(x, Wq, K_ext, V_ext, Wo):
    i = lax.axis_index("i")
    wq_s = lax.dynamic_slice(Wq, (0, i * DHID), (DM, DHID))
    wo_s = lax.dynamic_slice(Wo, (i * DHID, 0), (DHID, DM))

    wq_r = wq_s.reshape(DM, H_PER, DH).transpose(1, 0, 2)
    wo_r = wo_s.reshape(H_PER, DH, DM)
    k_r = K_ext.transpose(2, 0, 1, 3)
    v_r = V_ext.transpose(2, 0, 1, 3)
    x_f = x.reshape(ROWS, DM)

    out = pl.pallas_call(
        _fused_body,
        out_shape=jax.ShapeDtypeStruct((ROWS, DM), jnp.float32),
        in_specs=[pl.BlockSpec(memory_space=pltpu.VMEM)] * 5,
        out_specs=pl.BlockSpec(memory_space=pltpu.VMEM),
        scratch_shapes=[
            pltpu.VMEM((N_DEV, CHUNK, DM), jnp.float32),
            pltpu.VMEM((N_DEV, CHUNK, DM), jnp.float32),
            pltpu.VMEM((N_DEV, CHUNK, DM), jnp.float32),
            pltpu.VMEM((CHUNK, DM), jnp.float32),
            pltpu.SemaphoreType.DMA((N_DEV,)),
            pltpu.SemaphoreType.DMA((N_DEV,)),
            pltpu.SemaphoreType.DMA((N_DEV - 1,)),
            pltpu.SemaphoreType.DMA((N_DEV,)),
        ],
        compiler_params=pltpu.CompilerParams(collective_id=0),
    )(x_f, wq_r, k_r, v_r, wo_r)

    return out.reshape(B, SQ, DM)
